# baseline (speedup 1.0000x reference)
import math

import jax
import jax.numpy as jnp
import numpy as np
from jax.experimental import pallas as pl
from jax.experimental.pallas import tpu as pltpu

BB1 = 128

_NW = 15


def _body_pass1(xt_ref, yt_ref, *rest):
    weights = rest[:_NW]
    loss_ref, memk_out_ref = rest[_NW:_NW + 2]
    scratch = rest[_NW + 2:]
    _step_body(True, xt_ref, yt_ref, None, *weights,
               loss_ref=loss_ref, memk_out_ref=memk_out_ref, scratch=scratch)


def _body_pass2(xt_ref, yt_ref, memk_in_ref, *rest):
    weights = rest[:_NW]
    loss_ref = rest[_NW]
    scratch = rest[_NW + 1:]
    _step_body(False, xt_ref, yt_ref, memk_in_ref, *weights,
               loss_ref=loss_ref, memk_out_ref=None, scratch=scratch)


def _step_body(is_pass1,
               xt_ref, yt_ref, memk_in_ref,
               wx_ref, bx_ref, wh1_ref, bh1_ref, w2_ref, b2_ref,
               wrs_ref, brs_ref, wout_ref, bout_ref,
               cdft_ref, sdft_ref, hinit_ref, wrinit_ref, wwinit_ref,
               loss_ref=None, memk_out_ref=None, scratch=None):
    mem_scr, h_scr, wr_scr, ww_scr = scratch
    t = pl.program_id(1)
    bb, H = h_scr.shape
    M = wr_scr.shape[1]
    D = mem_scr.shape[1]

    @pl.when(t == 0)
    def _init():
        h_scr[...] = jnp.broadcast_to(hinit_ref[...], (bb, H))
        wr_scr[...] = jnp.broadcast_to(wrinit_ref[...], (bb, M))
        ww_scr[...] = jnp.broadcast_to(wwinit_ref[...], (bb, M))
        mem_scr[...] = jnp.zeros((bb, D, M), jnp.float32)

    h = h_scr[...]
    xt = xt_ref[0]
    yt = yt_ref[0]

    xp = jnp.dot(xt, wx_ref[...], preferred_element_type=jnp.float32) + bx_ref[...]
    hp = jnp.dot(h, wh1_ref[...], preferred_element_type=jnp.float32) + bh1_ref[...]
    z = jax.nn.sigmoid(xp[:, 0:H] + hp[:, 0:H])
    r = jax.nn.sigmoid(xp[:, H:2 * H] + hp[:, H:2 * H])
    cand = jnp.tanh(xp[:, 2 * H:3 * H] + hp[:, 2 * H:3 * H] * r)
    h = (1.0 - z) * h + z * cand

    p2 = jnp.dot(h, w2_ref[...], preferred_element_type=jnp.float32) + b2_ref[...]
    sr_log = p2[:, 0:M]
    sw_log = p2[:, M:2 * M]
    self_h = p2[:, 2 * M:2 * M + H]
    kr = jax.nn.relu(p2[:, 384:384 + D])
    kw = jax.nn.relu(p2[:, 404:404 + D])
    erase = p2[:, 424:424 + D]
    add = p2[:, 444:444 + D]
    betar = p2[:, 464:465]
    gr = p2[:, 465:466]
    betaw = p2[:, 466:467]
    gw = p2[:, 467:468]

    mem = mem_scr[...]
    if is_pass1:
        memkt = mem[bb - 1]
        memk_out_ref[0] = memkt
    else:
        memkt = memk_in_ref[0]

    keys2 = jnp.concatenate([kr, kw], axis=0)
    cos2 = jnp.dot(keys2, memkt, preferred_element_type=jnp.float32)
    beta2 = jnp.concatenate([betar, betaw], axis=0)
    g2 = jnp.concatenate([gr, gw], axis=0)
    prev2 = jnp.concatenate([wr_scr[...], ww_scr[...]], axis=0)
    slog2 = jnp.concatenate([sr_log, sw_log], axis=0)

    cb = cos2 * beta2
    cb = cb - jnp.max(cb, axis=-1, keepdims=True)
    e = jnp.exp(cb)
    w_content = e / jnp.sum(e, axis=-1, keepdims=True)
    wg2 = g2 * w_content + (1.0 - g2) * prev2

    sl = slog2 - jnp.max(slog2, axis=-1, keepdims=True)
    es = jnp.exp(sl)
    shift2 = es / jnp.sum(es, axis=-1, keepdims=True)

    C = cdft_ref[...]
    S = sdft_ref[...]
    xc = jnp.dot(wg2, C, preferred_element_type=jnp.float32)
    xs = jnp.dot(wg2, S, preferred_element_type=jnp.float32)
    yc = jnp.dot(shift2, C, preferred_element_type=jnp.float32)
    ys = jnp.dot(shift2, S, preferred_element_type=jnp.float32)
    rez = xc * yc - xs * ys
    imz_neg = xc * ys + xs * yc
    circ = (jnp.dot(rez, C, preferred_element_type=jnp.float32)
            + jnp.dot(imz_neg, S, preferred_element_type=jnp.float32)) * (1.0 / M)

    wp = jnp.exp(g2 * jnp.log(jnp.maximum(circ, 1e-12)))
    wnew2 = wp / jnp.sum(wp, axis=-1, keepdims=True)
    wr_new = wnew2[0:bb]
    ww_new = wnew2[bb:2 * bb]

    read = jnp.sum(wr_new[:, None, :] * mem, axis=2)

    mem = mem * (1.0 - ww_new[:, None, :] * erase[:, :, None]) \
        + ww_new[:, None, :] * add[:, :, None]
    mem_scr[...] = mem

    h = jnp.tanh(jnp.dot(read, wrs_ref[...], preferred_element_type=jnp.float32)
                 + brs_ref[...] + self_h)
    out = jnp.dot(h, wout_ref[...], preferred_element_type=jnp.float32) + bout_ref[...]
    step_loss = jnp.sum((out - yt) ** 2).reshape(1, 1, 1)

    h_scr[...] = h
    wr_scr[...] = wr_new
    ww_scr[...] = ww_new

    @pl.when(t == 0)
    def _loss0():
        loss_ref[...] = step_loss

    @pl.when(t != 0)
    def _lossacc():
        loss_ref[...] += step_loss


def _wspecs(weights):
    return [pl.BlockSpec(a.shape, lambda g, t, nd=a.ndim: (0,) * nd)
            for a in weights]


def _scratch(bb, D, M, H):
    f32 = jnp.float32
    return [
        pltpu.VMEM((bb, D, M), f32),
        pltpu.VMEM((bb, H), f32),
        pltpu.VMEM((bb, M), f32),
        pltpu.VMEM((bb, M), f32),
    ]


def _pass1(xK, yK, weights, D, M, H):
    T, bb, IN = xK.shape
    f32 = jnp.float32
    return pl.pallas_call(
        _body_pass1,
        grid=(1, T),
        in_specs=[
            pl.BlockSpec((1, bb, IN), lambda g, t: (t, 0, 0)),
            pl.BlockSpec((1, bb, IN), lambda g, t: (t, 0, 0)),
        ] + _wspecs(weights),
        out_specs=[
            pl.BlockSpec((1, 1, 1), lambda g, t: (0, 0, 0)),
            pl.BlockSpec((1, D, M), lambda g, t: (t, 0, 0)),
        ],
        out_shape=[
            jax.ShapeDtypeStruct((1, 1, 1), f32),
            jax.ShapeDtypeStruct((T, D, M), f32),
        ],
        scratch_shapes=_scratch(bb, D, M, H),
        compiler_params=pltpu.CompilerParams(
            dimension_semantics=("arbitrary", "arbitrary")),
    )(xK, yK, *weights)


def _pass2(xT, yT, memk, weights, bb, D, M, H):
    T, N, IN = xT.shape
    G = N // bb
    f32 = jnp.float32
    (loss2,) = pl.pallas_call(
        _body_pass2,
        grid=(G, T),
        in_specs=[
            pl.BlockSpec((1, bb, IN), lambda g, t: (t, g, 0)),
            pl.BlockSpec((1, bb, IN), lambda g, t: (t, g, 0)),
            pl.BlockSpec((1, D, M), lambda g, t: (t, 0, 0)),
        ] + _wspecs(weights),
        out_specs=[
            pl.BlockSpec((1, 1, 1), lambda g, t: (g, 0, 0)),
        ],
        out_shape=[jax.ShapeDtypeStruct((G, 1, 1), f32)],
        scratch_shapes=_scratch(bb, D, M, H),
        compiler_params=pltpu.CompilerParams(
            dimension_semantics=("parallel", "arbitrary")),
    )(xT, yT, memk, *weights)
    return loss2


def kernel(x, y, params):
    B, T, IN = x.shape
    p = params
    H = p['h_init'].shape[0]
    M = p['rw_init'].shape[0]
    D = p['read']['k'][0].shape[0]
    f32 = jnp.float32

    wx = jnp.concatenate([p['wx_update'][0].T, p['wx_reset'][0].T,
                          p['wx_hidden'][0].T], axis=1)
    bx = jnp.concatenate([p['wx_update'][1] + p['wh_update'][1],
                          p['wx_reset'][1] + p['wh_reset'][1],
                          p['wx_hidden'][1]])[None, :]
    wh1 = jnp.concatenate([p['wh_update'][0].T, p['wh_reset'][0].T,
                           p['wh_hidden'][0].T], axis=1)
    bh1 = jnp.concatenate([jnp.zeros((2 * H,), f32),
                           p['wh_hidden'][1]])[None, :]

    rp, wp_ = p['read'], p['write']
    w2 = jnp.concatenate([
        rp['s'][0].T, wp_['s'][0].T, p['self_sec'][0].T,
        rp['k'][0].T, wp_['k'][0].T, wp_['erase'][0].T, wp_['add'][0].T,
        rp['beta'][0].T, rp['g'][0].T, wp_['beta'][0].T, wp_['g'][0].T,
        jnp.zeros((H, 44), f32)], axis=1)
    b2 = jnp.concatenate([
        rp['s'][1], wp_['s'][1], p['self_sec'][1],
        rp['k'][1], wp_['k'][1], wp_['erase'][1], wp_['add'][1],
        rp['beta'][1], rp['g'][1], wp_['beta'][1], wp_['g'][1],
        jnp.zeros((44,), f32)])[None, :]

    wrs = p['read_sec'][0].T
    brs = p['read_sec'][1][None, :]
    wout = p['out'][0].T
    bout = p['out'][1][None, :]

    jk = np.arange(M, dtype=np.float64)
    ang = 2.0 * math.pi * np.outer(jk, jk) / M
    cdft = jnp.asarray(np.cos(ang), f32)
    sdft = jnp.asarray(np.sin(ang), f32)

    hinit = p['h_init'][None, :]
    wrinit = p['rw_init'][None, :]
    wwinit = p['ww_init'][None, :]

    xT = jnp.swapaxes(x, 0, 1)
    yT = jnp.swapaxes(y, 0, 1)
    xK, yK = xT[:, B - BB1:], yT[:, B - BB1:]
    x2, y2 = xT[:, :B - BB1], yT[:, :B - BB1]
    N2 = B - BB1

    weights = (wx, bx, wh1, bh1, w2, b2, wrs, brs, wout, bout,
               cdft, sdft, hinit, wrinit, wwinit)

    devs = jax.devices()
    if len(devs) >= 2 and N2 % 2 == 0:
        mesh = jax.sharding.Mesh(np.asarray(devs[:2]), ('b',))
        P = jax.sharding.PartitionSpec
        try:
            smap = jax.shard_map
        except AttributeError:
            from jax.experimental.shard_map import shard_map as smap

        def shard_fn(x2l, y2l, xKl, yKl, *ws):
            loss1, memk = _pass1(xKl, yKl, ws, D, M, H)
            loss2 = _pass2(x2l, y2l, memk, ws, x2l.shape[1], D, M, H)
            idx = jax.lax.axis_index('b')
            l = jnp.sum(loss2) + jnp.where(idx == 0, jnp.sum(loss1), 0.0)
            return jax.lax.psum(l, 'b')

        return smap(
            shard_fn, mesh=mesh,
            in_specs=(P(None, 'b', None), P(None, 'b', None),
                      P(), P()) + tuple(P() for _ in weights),
            out_specs=P(), check_vma=False,
        )(x2, y2, xK, yK, *weights)

    loss1, memk = _pass1(xK, yK, weights, D, M, H)
    loss2 = _pass2(x2, y2, memk, weights, N2 // 2, D, M, H)
    return jnp.sum(loss1) + jnp.sum(loss2)


# speedup vs baseline: 1.9562x; 1.9562x over previous
import math

import jax
import jax.numpy as jnp
import numpy as np
from jax.experimental import pallas as pl
from jax.experimental.pallas import tpu as pltpu

BB1 = 512

_NW = 15


def _body_pass1(xt_ref, yt_ref, *rest):
    weights = rest[:_NW]
    loss_ref, memk_out_ref = rest[_NW:_NW + 2]
    scratch = rest[_NW + 2:]
    _step_body(True, xt_ref, yt_ref, None, *weights,
               loss_ref=loss_ref, memk_out_ref=memk_out_ref, scratch=scratch)


def _body_pass2(xt_ref, yt_ref, memk_in_ref, *rest):
    weights = rest[:_NW]
    loss_ref = rest[_NW]
    scratch = rest[_NW + 1:]
    _step_body(False, xt_ref, yt_ref, memk_in_ref, *weights,
               loss_ref=loss_ref, memk_out_ref=None, scratch=scratch)


def _step_body(is_pass1,
               xt_ref, yt_ref, memk_in_ref,
               wx_ref, bx_ref, wh1_ref, bh1_ref, w2_ref, b2_ref,
               wrs_ref, brs_ref, wout_ref, bout_ref,
               cdft_ref, sdft_ref, hinit_ref, wrinit_ref, wwinit_ref,
               loss_ref=None, memk_out_ref=None, scratch=None):
    mem_scr, h_scr, wr_scr, ww_scr = scratch
    t = pl.program_id(1)
    bb, H = h_scr.shape
    M = wr_scr.shape[1]
    D = mem_scr.shape[0]

    @pl.when(t == 0)
    def _init():
        h_scr[...] = jnp.broadcast_to(hinit_ref[...], (bb, H))
        wr_scr[...] = jnp.broadcast_to(wrinit_ref[...], (bb, M))
        ww_scr[...] = jnp.broadcast_to(wwinit_ref[...], (bb, M))
        mem_scr[...] = jnp.zeros((D, bb, M), jnp.float32)

    h = h_scr[...]
    xt = xt_ref[0]
    yt = yt_ref[0]

    xp = jnp.dot(xt, wx_ref[...], preferred_element_type=jnp.float32) + bx_ref[...]
    hp = jnp.dot(h, wh1_ref[...], preferred_element_type=jnp.float32) + bh1_ref[...]
    z = jax.nn.sigmoid(xp[:, 0:H] + hp[:, 0:H])
    r = jax.nn.sigmoid(xp[:, H:2 * H] + hp[:, H:2 * H])
    cand = jnp.tanh(xp[:, 2 * H:3 * H] + hp[:, 2 * H:3 * H] * r)
    h = (1.0 - z) * h + z * cand

    p2 = jnp.dot(h, w2_ref[...], preferred_element_type=jnp.float32) + b2_ref[...]
    sr_log = p2[:, 0:M]
    sw_log = p2[:, M:2 * M]
    self_h = p2[:, 2 * M:2 * M + H]
    kr = jax.nn.relu(p2[:, 384:384 + D])
    kw = jax.nn.relu(p2[:, 404:404 + D])
    erase = p2[:, 424:424 + D]
    add = p2[:, 444:444 + D]
    betar = p2[:, 464:465]
    gr = p2[:, 465:466]
    betaw = p2[:, 466:467]
    gw = p2[:, 467:468]

    mem = mem_scr[...]
    if is_pass1:
        memkt = mem[:, bb - 1, :]
        memk_out_ref[0] = memkt
    else:
        memkt = memk_in_ref[0]

    keys2 = jnp.concatenate([kr, kw], axis=0)
    cos2 = jnp.dot(keys2, memkt, preferred_element_type=jnp.float32)
    beta2 = jnp.concatenate([betar, betaw], axis=0)
    g2 = jnp.concatenate([gr, gw], axis=0)
    prev2 = jnp.concatenate([wr_scr[...], ww_scr[...]], axis=0)
    slog2 = jnp.concatenate([sr_log, sw_log], axis=0)

    cb = cos2 * beta2
    cb = cb - jnp.max(cb, axis=-1, keepdims=True)
    e = jnp.exp(cb)
    w_content = e / jnp.sum(e, axis=-1, keepdims=True)
    wg2 = g2 * w_content + (1.0 - g2) * prev2

    sl = slog2 - jnp.max(slog2, axis=-1, keepdims=True)
    es = jnp.exp(sl)
    shift2 = es / jnp.sum(es, axis=-1, keepdims=True)

    C = cdft_ref[...]
    S = sdft_ref[...]
    xc = jnp.dot(wg2, C, preferred_element_type=jnp.float32)
    xs = jnp.dot(wg2, S, preferred_element_type=jnp.float32)
    yc = jnp.dot(shift2, C, preferred_element_type=jnp.float32)
    ys = jnp.dot(shift2, S, preferred_element_type=jnp.float32)
    rez = xc * yc - xs * ys
    imz_neg = xc * ys + xs * yc
    circ = (jnp.dot(rez, C, preferred_element_type=jnp.float32)
            + jnp.dot(imz_neg, S, preferred_element_type=jnp.float32)) * (1.0 / M)

    wp = jnp.exp(g2 * jnp.log(jnp.maximum(circ, 1e-12)))
    wnew2 = wp / jnp.sum(wp, axis=-1, keepdims=True)
    wr_new = wnew2[0:bb]
    ww_new = wnew2[bb:2 * bb]

    ones_m = jnp.ones((M, 1), jnp.float32)
    read_cols = []
    new_slabs = []
    for d in range(D):
        slab = mem[d]
        read_cols.append(jnp.dot(wr_new * slab, ones_m,
                                 preferred_element_type=jnp.float32))
        scale = 1.0 - ww_new * erase[:, d:d + 1]
        new_slabs.append(slab * scale + ww_new * add[:, d:d + 1])
    read = jnp.concatenate(read_cols, axis=1)
    mem_scr[...] = jnp.stack(new_slabs, axis=0)

    h = jnp.tanh(jnp.dot(read, wrs_ref[...], preferred_element_type=jnp.float32)
                 + brs_ref[...] + self_h)
    out = jnp.dot(h, wout_ref[...], preferred_element_type=jnp.float32) + bout_ref[...]
    step_loss = jnp.sum((out - yt) ** 2).reshape(1, 1, 1)

    h_scr[...] = h
    wr_scr[...] = wr_new
    ww_scr[...] = ww_new

    @pl.when(t == 0)
    def _loss0():
        loss_ref[...] = step_loss

    @pl.when(t != 0)
    def _lossacc():
        loss_ref[...] += step_loss


def _wspecs(weights):
    return [pl.BlockSpec(a.shape, lambda g, t, nd=a.ndim: (0,) * nd)
            for a in weights]


def _scratch(bb, D, M, H):
    f32 = jnp.float32
    return [
        pltpu.VMEM((D, bb, M), f32),
        pltpu.VMEM((bb, H), f32),
        pltpu.VMEM((bb, M), f32),
        pltpu.VMEM((bb, M), f32),
    ]


def _pass1(xK, yK, weights, D, M, H):
    T, bb, IN = xK.shape
    f32 = jnp.float32
    return pl.pallas_call(
        _body_pass1,
        grid=(1, T),
        in_specs=[
            pl.BlockSpec((1, bb, IN), lambda g, t: (t, 0, 0)),
            pl.BlockSpec((1, bb, IN), lambda g, t: (t, 0, 0)),
        ] + _wspecs(weights),
        out_specs=[
            pl.BlockSpec((1, 1, 1), lambda g, t: (0, 0, 0)),
            pl.BlockSpec((1, D, M), lambda g, t: (t, 0, 0)),
        ],
        out_shape=[
            jax.ShapeDtypeStruct((1, 1, 1), f32),
            jax.ShapeDtypeStruct((T, D, M), f32),
        ],
        scratch_shapes=_scratch(bb, D, M, H),
        compiler_params=pltpu.CompilerParams(
            dimension_semantics=("arbitrary", "arbitrary")),
    )(xK, yK, *weights)


def _pass2(xT, yT, memk, weights, bb, D, M, H):
    T, N, IN = xT.shape
    G = N // bb
    f32 = jnp.float32
    (loss2,) = pl.pallas_call(
        _body_pass2,
        grid=(G, T),
        in_specs=[
            pl.BlockSpec((1, bb, IN), lambda g, t: (t, g, 0)),
            pl.BlockSpec((1, bb, IN), lambda g, t: (t, g, 0)),
            pl.BlockSpec((1, D, M), lambda g, t: (t, 0, 0)),
        ] + _wspecs(weights),
        out_specs=[
            pl.BlockSpec((1, 1, 1), lambda g, t: (g, 0, 0)),
        ],
        out_shape=[jax.ShapeDtypeStruct((G, 1, 1), f32)],
        scratch_shapes=_scratch(bb, D, M, H),
        compiler_params=pltpu.CompilerParams(
            dimension_semantics=("parallel", "arbitrary")),
    )(xT, yT, memk, *weights)
    return loss2


def kernel(x, y, params):
    B, T, IN = x.shape
    p = params
    H = p['h_init'].shape[0]
    M = p['rw_init'].shape[0]
    D = p['read']['k'][0].shape[0]
    f32 = jnp.float32

    wx = jnp.concatenate([p['wx_update'][0].T, p['wx_reset'][0].T,
                          p['wx_hidden'][0].T], axis=1)
    bx = jnp.concatenate([p['wx_update'][1] + p['wh_update'][1],
                          p['wx_reset'][1] + p['wh_reset'][1],
                          p['wx_hidden'][1]])[None, :]
    wh1 = jnp.concatenate([p['wh_update'][0].T, p['wh_reset'][0].T,
                           p['wh_hidden'][0].T], axis=1)
    bh1 = jnp.concatenate([jnp.zeros((2 * H,), f32),
                           p['wh_hidden'][1]])[None, :]

    rp, wp_ = p['read'], p['write']
    w2 = jnp.concatenate([
        rp['s'][0].T, wp_['s'][0].T, p['self_sec'][0].T,
        rp['k'][0].T, wp_['k'][0].T, wp_['erase'][0].T, wp_['add'][0].T,
        rp['beta'][0].T, rp['g'][0].T, wp_['beta'][0].T, wp_['g'][0].T,
        jnp.zeros((H, 44), f32)], axis=1)
    b2 = jnp.concatenate([
        rp['s'][1], wp_['s'][1], p['self_sec'][1],
        rp['k'][1], wp_['k'][1], wp_['erase'][1], wp_['add'][1],
        rp['beta'][1], rp['g'][1], wp_['beta'][1], wp_['g'][1],
        jnp.zeros((44,), f32)])[None, :]

    wrs = p['read_sec'][0].T
    brs = p['read_sec'][1][None, :]
    wout = p['out'][0].T
    bout = p['out'][1][None, :]

    jk = np.arange(M, dtype=np.float64)
    ang = 2.0 * math.pi * np.outer(jk, jk) / M
    cdft = jnp.asarray(np.cos(ang), f32)
    sdft = jnp.asarray(np.sin(ang), f32)

    hinit = p['h_init'][None, :]
    wrinit = p['rw_init'][None, :]
    wwinit = p['ww_init'][None, :]

    xT = jnp.swapaxes(x, 0, 1)
    yT = jnp.swapaxes(y, 0, 1)
    bb1 = min(BB1, B // 2)
    xK, yK = xT[:, B - bb1:], yT[:, B - bb1:]
    x2, y2 = xT[:, :B - bb1], yT[:, :B - bb1]
    N2 = B - bb1

    weights = (wx, bx, wh1, bh1, w2, b2, wrs, brs, wout, bout,
               cdft, sdft, hinit, wrinit, wwinit)

    loss1, memk = _pass1(xK, yK, weights, D, M, H)
    loss2 = _pass2(x2, y2, memk, weights, N2, D, M, H)
    return jnp.sum(loss1) + jnp.sum(loss2)


# speedup vs baseline: 2.0654x; 1.0558x over previous
import math

import jax
import jax.numpy as jnp
import numpy as np
from jax.experimental import pallas as pl
from jax.experimental.pallas import tpu as pltpu

BB1 = 512

_NW = 15


def _body_fused(xt_ref, yt_ref, *rest):
    weights = rest[:_NW]
    loss_ref = rest[_NW]
    scratch = rest[_NW + 1:]
    _step_body(True, xt_ref, yt_ref, None, *weights,
               loss_ref=loss_ref, memk_out_ref=None, scratch=scratch)


def _step_body(is_pass1,
               xt_ref, yt_ref, memk_in_ref,
               wx_ref, bx_ref, wh1_ref, bh1_ref, w2_ref, b2_ref,
               wrs_ref, brs_ref, wout_ref, bout_ref,
               cdft_ref, sdft_ref, hinit_ref, wrinit_ref, wwinit_ref,
               loss_ref=None, memk_out_ref=None, scratch=None):
    mem_scr, h_scr, wr_scr, ww_scr = scratch
    t = pl.program_id(1)
    bb, H = h_scr.shape
    M = wr_scr.shape[1]
    D = mem_scr.shape[0]

    @pl.when(t == 0)
    def _init():
        h_scr[...] = jnp.broadcast_to(hinit_ref[...], (bb, H))
        wr_scr[...] = jnp.broadcast_to(wrinit_ref[...], (bb, M))
        ww_scr[...] = jnp.broadcast_to(wwinit_ref[...], (bb, M))
        mem_scr[...] = jnp.zeros((D, bb, M), jnp.float32)

    h = h_scr[...]
    xt = xt_ref[0]
    yt = yt_ref[0]

    xp = jnp.dot(xt, wx_ref[...], preferred_element_type=jnp.float32) + bx_ref[...]
    hp = jnp.dot(h, wh1_ref[...], preferred_element_type=jnp.float32) + bh1_ref[...]
    z = jax.nn.sigmoid(xp[:, 0:H] + hp[:, 0:H])
    r = jax.nn.sigmoid(xp[:, H:2 * H] + hp[:, H:2 * H])
    cand = jnp.tanh(xp[:, 2 * H:3 * H] + hp[:, 2 * H:3 * H] * r)
    h = (1.0 - z) * h + z * cand

    p2 = jnp.dot(h, w2_ref[...], preferred_element_type=jnp.float32) + b2_ref[...]
    sr_log = p2[:, 0:M]
    sw_log = p2[:, M:2 * M]
    self_h = p2[:, 2 * M:2 * M + H]
    kr = jax.nn.relu(p2[:, 384:384 + D])
    kw = jax.nn.relu(p2[:, 404:404 + D])
    erase = p2[:, 424:424 + D]
    add = p2[:, 444:444 + D]
    betar = p2[:, 464:465]
    gr = p2[:, 465:466]
    betaw = p2[:, 466:467]
    gw = p2[:, 467:468]

    mem = mem_scr[...]
    memkt = mem[:, bb - 1, :]

    keys2 = jnp.concatenate([kr, kw], axis=0)
    cos2 = jnp.dot(keys2, memkt, preferred_element_type=jnp.float32)
    beta2 = jnp.concatenate([betar, betaw], axis=0)
    g2 = jnp.concatenate([gr, gw], axis=0)
    prev2 = jnp.concatenate([wr_scr[...], ww_scr[...]], axis=0)
    slog2 = jnp.concatenate([sr_log, sw_log], axis=0)

    cb = cos2 * beta2
    cb = cb - jnp.max(cb, axis=-1, keepdims=True)
    e = jnp.exp(cb)
    w_content = e / jnp.sum(e, axis=-1, keepdims=True)
    wg2 = g2 * w_content + (1.0 - g2) * prev2

    sl = slog2 - jnp.max(slog2, axis=-1, keepdims=True)
    es = jnp.exp(sl)
    shift2 = es / jnp.sum(es, axis=-1, keepdims=True)

    C = cdft_ref[...]
    S = sdft_ref[...]
    xc = jnp.dot(wg2, C, preferred_element_type=jnp.float32)
    xs = jnp.dot(wg2, S, preferred_element_type=jnp.float32)
    yc = jnp.dot(shift2, C, preferred_element_type=jnp.float32)
    ys = jnp.dot(shift2, S, preferred_element_type=jnp.float32)
    rez = xc * yc - xs * ys
    imz_neg = xc * ys + xs * yc
    circ = (jnp.dot(rez, C, preferred_element_type=jnp.float32)
            + jnp.dot(imz_neg, S, preferred_element_type=jnp.float32)) * (1.0 / M)

    wp = jnp.exp(g2 * jnp.log(jnp.maximum(circ, 1e-12)))
    wnew2 = wp / jnp.sum(wp, axis=-1, keepdims=True)
    wr_new = wnew2[0:bb]
    ww_new = wnew2[bb:2 * bb]

    ones_m = jnp.ones((M, 1), jnp.float32)
    read_cols = []
    new_slabs = []
    for d in range(D):
        slab = mem[d]
        read_cols.append(jnp.dot(wr_new * slab, ones_m,
                                 preferred_element_type=jnp.float32))
        scale = 1.0 - ww_new * erase[:, d:d + 1]
        new_slabs.append(slab * scale + ww_new * add[:, d:d + 1])
    read = jnp.concatenate(read_cols, axis=1)
    mem_scr[...] = jnp.stack(new_slabs, axis=0)

    h = jnp.tanh(jnp.dot(read, wrs_ref[...], preferred_element_type=jnp.float32)
                 + brs_ref[...] + self_h)
    out = jnp.dot(h, wout_ref[...], preferred_element_type=jnp.float32) + bout_ref[...]
    step_loss = jnp.sum((out - yt) ** 2).reshape(1, 1, 1)

    h_scr[...] = h
    wr_scr[...] = wr_new
    ww_scr[...] = ww_new

    @pl.when(t == 0)
    def _loss0():
        loss_ref[...] = step_loss

    @pl.when(t != 0)
    def _lossacc():
        loss_ref[...] += step_loss


def _wspecs(weights):
    return [pl.BlockSpec(a.shape, lambda g, t, nd=a.ndim: (0,) * nd)
            for a in weights]


def _scratch(bb, D, M, H):
    f32 = jnp.float32
    return [
        pltpu.VMEM((D, bb, M), f32),
        pltpu.VMEM((bb, H), f32),
        pltpu.VMEM((bb, M), f32),
        pltpu.VMEM((bb, M), f32),
    ]


def _fused(xT, yT, weights, D, M, H):
    T, bb, IN = xT.shape
    f32 = jnp.float32
    (loss,) = pl.pallas_call(
        _body_fused,
        grid=(1, T),
        in_specs=[
            pl.BlockSpec((1, bb, IN), lambda g, t: (t, 0, 0)),
            pl.BlockSpec((1, bb, IN), lambda g, t: (t, 0, 0)),
        ] + _wspecs(weights),
        out_specs=[
            pl.BlockSpec((1, 1, 1), lambda g, t: (0, 0, 0)),
        ],
        out_shape=[jax.ShapeDtypeStruct((1, 1, 1), f32)],
        scratch_shapes=_scratch(bb, D, M, H),
        compiler_params=pltpu.CompilerParams(
            dimension_semantics=("arbitrary", "arbitrary")),
    )(xT, yT, *weights)
    return loss


def kernel(x, y, params):
    B, T, IN = x.shape
    p = params
    H = p['h_init'].shape[0]
    M = p['rw_init'].shape[0]
    D = p['read']['k'][0].shape[0]
    f32 = jnp.float32

    wx = jnp.concatenate([p['wx_update'][0].T, p['wx_reset'][0].T,
                          p['wx_hidden'][0].T], axis=1)
    bx = jnp.concatenate([p['wx_update'][1] + p['wh_update'][1],
                          p['wx_reset'][1] + p['wh_reset'][1],
                          p['wx_hidden'][1]])[None, :]
    wh1 = jnp.concatenate([p['wh_update'][0].T, p['wh_reset'][0].T,
                           p['wh_hidden'][0].T], axis=1)
    bh1 = jnp.concatenate([jnp.zeros((2 * H,), f32),
                           p['wh_hidden'][1]])[None, :]

    rp, wp_ = p['read'], p['write']
    w2 = jnp.concatenate([
        rp['s'][0].T, wp_['s'][0].T, p['self_sec'][0].T,
        rp['k'][0].T, wp_['k'][0].T, wp_['erase'][0].T, wp_['add'][0].T,
        rp['beta'][0].T, rp['g'][0].T, wp_['beta'][0].T, wp_['g'][0].T,
        jnp.zeros((H, 44), f32)], axis=1)
    b2 = jnp.concatenate([
        rp['s'][1], wp_['s'][1], p['self_sec'][1],
        rp['k'][1], wp_['k'][1], wp_['erase'][1], wp_['add'][1],
        rp['beta'][1], rp['g'][1], wp_['beta'][1], wp_['g'][1],
        jnp.zeros((44,), f32)])[None, :]

    wrs = p['read_sec'][0].T
    brs = p['read_sec'][1][None, :]
    wout = p['out'][0].T
    bout = p['out'][1][None, :]

    jk = np.arange(M, dtype=np.float64)
    ang = 2.0 * math.pi * np.outer(jk, jk) / M
    cdft = jnp.asarray(np.cos(ang), f32)
    sdft = jnp.asarray(np.sin(ang), f32)

    hinit = p['h_init'][None, :]
    wrinit = p['rw_init'][None, :]
    wwinit = p['ww_init'][None, :]

    xT = jnp.swapaxes(x, 0, 1)
    yT = jnp.swapaxes(y, 0, 1)
    weights = (wx, bx, wh1, bh1, w2, b2, wrs, brs, wout, bout,
               cdft, sdft, hinit, wrinit, wwinit)

    loss = _fused(xT, yT, weights, D, M, H)
    return jnp.sum(loss)
